# baseline (speedup 1.0000x reference)
import functools

import jax
import jax.numpy as jnp
from jax import lax
from jax.experimental import pallas as pl
from jax.experimental.pallas import tpu as pltpu
from jax.experimental.pallas import tpu_sc as plsc

B = 16384
E = 32
L = 16
W = 128
RPB = W // E

_info = plsc.get_sparse_core_info()
_NC, _NS = _info.num_cores, _info.num_subcores
NW = _NC * _NS
BPW = B // NW
CHUNK = 256
NCHUNK = BPW // CHUNK


def _sc_kernel(user_hbm, item_hbm, ut_hbm, it_hbm, out_hbm,
               uidx_v, iidx_v, ublk_v, iblk_v, urow_v, irow_v, out_v,
               sem_u, sem_i):
    wid = lax.axis_index("s") * _NC + lax.axis_index("c")
    base = wid * BPW
    pltpu.sync_copy(user_hbm.at[pl.ds(base, BPW)], uidx_v)
    pltpu.sync_copy(item_hbm.at[pl.ds(base, BPW)], iidx_v)
    lanes = lax.iota(jnp.int32, L)

    def to_blocks(g, carry):
        ublk_v[pl.ds(g * L, L)] = uidx_v[pl.ds(g * L, L)] >> 2
        iblk_v[pl.ds(g * L, L)] = iidx_v[pl.ds(g * L, L)] >> 2
        return carry

    lax.fori_loop(0, BPW // L, to_blocks, 0)

    for c in range(NCHUNK):
        c0 = c * CHUNK
        cp_u = pltpu.async_copy(
            ut_hbm.at[ublk_v.at[pl.ds(c0, CHUNK)]], urow_v, sem_u)
        cp_i = pltpu.async_copy(
            it_hbm.at[iblk_v.at[pl.ds(c0, CHUNK)]], irow_v, sem_i)
        cp_u.wait()
        cp_i.wait()

        def compute(g, carry):
            rows = g * L + lanes
            ucol = (uidx_v[pl.ds(c0 + g * L, L)] & 3) << 5
            icol = (iidx_v[pl.ds(c0 + g * L, L)] & 3) << 5
            acc = (plsc.load_gather(urow_v, [rows, ucol])
                   * plsc.load_gather(irow_v, [rows, icol]))
            for e in range(1, E):
                acc = acc + (plsc.load_gather(urow_v, [rows, ucol + e])
                             * plsc.load_gather(irow_v, [rows, icol + e]))
            out_v[pl.ds(c0 + g * L, L)] = acc
            return carry

        lax.fori_loop(0, CHUNK // L, compute, 0)

    pltpu.sync_copy(out_v, out_hbm.at[pl.ds(base, BPW)])


@jax.jit
def kernel(user, item, user_table, item_table):
    user = user.astype(jnp.int32)
    item = item.astype(jnp.int32)
    ut2 = user_table.reshape(250000, W)
    it2 = item_table.reshape(250000, W)
    mesh = plsc.VectorSubcoreMesh(core_axis_name="c", subcore_axis_name="s")
    f = functools.partial(
        pl.kernel,
        mesh=mesh,
        out_type=jax.ShapeDtypeStruct((B,), jnp.float32),
        compiler_params=pltpu.CompilerParams(needs_layout_passes=False),
        scratch_types=[
            pltpu.VMEM((BPW,), jnp.int32),
            pltpu.VMEM((BPW,), jnp.int32),
            pltpu.VMEM((BPW,), jnp.int32),
            pltpu.VMEM((BPW,), jnp.int32),
            pltpu.VMEM((CHUNK, W), jnp.float32),
            pltpu.VMEM((CHUNK, W), jnp.float32),
            pltpu.VMEM((BPW,), jnp.float32),
            pltpu.SemaphoreType.DMA,
            pltpu.SemaphoreType.DMA,
        ],
    )(_sc_kernel)
    return f(user, item, ut2, it2)
